# baseline (speedup 1.0000x reference)
import jax
import jax.numpy as jnp
from jax import lax
from jax.experimental import pallas as pl
from jax.experimental.pallas import tpu as pltpu
from jax.experimental.pallas import tpu_sc as plsc

N_NODES = 10000
D_FEAT = 128
OUT_NODES = 100
N_EDGES = 320000

NC = 2
NS = 16
NW = NC * NS
CHUNK = 128
EPW = (N_EDGES + NW - 1) // NW
CPW = 80
HALF = CPW // 2
EPW_PAD = CPW * CHUNK
E_PAD = NW * EPW_PAD
ACC_ROWS = NS * 640
DUMMY = N_NODES
RPT = ACC_ROWS // NS
DEG_W = 8


def _sc_mesh():
    return plsc.VectorSubcoreMesh(
        core_axis_name="c", subcore_axis_name="s", num_cores=NC, num_subcores=NS
    )


def _deg_body(dst_hbm, out_hbm, dst_v, ones_v, acc, sem):
    c = lax.axis_index("c")
    s = lax.axis_index("s")
    wid = c * NS + s

    zeros16 = jnp.zeros((16,), jnp.float32)
    ones16 = jnp.ones((16,), jnp.float32)

    @pl.loop(0, CHUNK)
    def _(r):
        for k in range(D_FEAT // 16):
            ones_v[r, pl.ds(k * 16, 16)] = zeros16

    pltpu.sync_copy(dst_hbm.at[wid], dst_v)
    for h in range(RPT // CHUNK):
        pltpu.sync_copy(ones_v, acc.at[pl.ds(s * RPT + h * CHUNK, CHUNK)])

    @pl.loop(0, CHUNK)
    def _(r):
        for k in range(D_FEAT // 16):
            ones_v[r, pl.ds(k * 16, 16)] = ones16

    plsc.subcore_barrier()

    @pl.loop(0, 8)
    def _(j):
        pltpu.async_copy(ones_v, acc.at[dst_v.at[j]], sem, add=True)

    @pl.loop(8, CPW)
    def _(j):
        pltpu.async_copy(ones_v, acc.at[dst_v.at[j]], sem, add=True)
        pltpu.make_async_copy(out_hbm.at[c, pl.ds(0, CHUNK)], ones_v, sem).wait()

    @pl.loop(0, 8)
    def _(j):
        pltpu.make_async_copy(out_hbm.at[c, pl.ds(0, CHUNK)], ones_v, sem).wait()

    plsc.subcore_barrier()
    for h in range(RPT // CHUNK):
        pltpu.sync_copy(acc.at[pl.ds(s * RPT + h * CHUNK, CHUNK)], ones_v)
        pltpu.sync_copy(ones_v, out_hbm.at[c, pl.ds(s * RPT + h * CHUNK, CHUNK)])


def _make_deg_kernel():
    return pl.kernel(
        _deg_body,
        out_type=jax.ShapeDtypeStruct((NC, ACC_ROWS, D_FEAT), jnp.float32),
        mesh=_sc_mesh(),
        scratch_types=[
            pltpu.VMEM((CPW, CHUNK), jnp.int32),
            pltpu.VMEM((CHUNK, D_FEAT), jnp.float32),
            pltpu.VMEM_SHARED((ACC_ROWS, D_FEAT), jnp.float32),
            pltpu.SemaphoreType.DMA,
        ],
    )


def _agg_body(g_hbm, src_hbm, dst_hbm, out_hbm, src_v, dst_v, ra, rb,
              acc, gsa, gsb, ssa, ssb):
    c = lax.axis_index("c")
    s = lax.axis_index("s")
    wid = c * NS + s

    zeros16 = jnp.zeros((16,), jnp.float32)

    @pl.loop(0, CHUNK)
    def _(r):
        for k in range(D_FEAT // 16):
            ra[r, pl.ds(k * 16, 16)] = zeros16

    for h in range(RPT // CHUNK):
        pltpu.sync_copy(ra, acc.at[pl.ds(s * RPT + h * CHUNK, CHUNK)])
    plsc.subcore_barrier()

    def wait_gather(buf, sem):
        pltpu.make_async_copy(g_hbm.at[src_v.at[0]], buf, sem).wait()

    def wait_scatter(buf, sem):
        pltpu.make_async_copy(g_hbm.at[src_v.at[0]], buf, sem).wait()

    for half in range(2):
        pltpu.sync_copy(src_hbm.at[wid, pl.ds(half * HALF, HALF)], src_v)
        pltpu.sync_copy(dst_hbm.at[wid, pl.ds(half * HALF, HALF)], dst_v)

        @pl.loop(0, HALF)
        def _(j):
            pltpu.async_copy(g_hbm.at[src_v.at[j]], ra, gsa).wait()
            pltpu.async_copy(ra, acc.at[dst_v.at[j]], ssa, add=True).wait()

    plsc.subcore_barrier()
    for h in range(RPT // CHUNK):
        pltpu.sync_copy(acc.at[pl.ds(s * RPT + h * CHUNK, CHUNK)], ra)
        pltpu.sync_copy(ra, out_hbm.at[c, pl.ds(s * RPT + h * CHUNK, CHUNK)])


def _make_agg_kernel():
    return pl.kernel(
        _agg_body,
        out_type=jax.ShapeDtypeStruct((NC, ACC_ROWS, D_FEAT), jnp.float32),
        mesh=_sc_mesh(),
        scratch_types=[
            pltpu.VMEM((HALF, CHUNK), jnp.int32),
            pltpu.VMEM((HALF, CHUNK), jnp.int32),
            pltpu.VMEM((CHUNK, D_FEAT), jnp.float32),
            pltpu.VMEM((CHUNK, D_FEAT), jnp.float32),
            pltpu.VMEM_SHARED((ACC_ROWS, D_FEAT), jnp.float32),
            pltpu.SemaphoreType.DMA,
            pltpu.SemaphoreType.DMA,
            pltpu.SemaphoreType.DMA,
            pltpu.SemaphoreType.DMA,
        ],
    )


def _tc1_body(x_ref, w_ref, da_ref, db_ref, g_ref):
    p = lax.rsqrt(da_ref[:, 0:1] + db_ref[:, 0:1] + 1.0)
    h = jnp.dot(x_ref[...], w_ref[...], preferred_element_type=jnp.float32)
    g_ref[...] = p * h


def _tc2_body(pa_ref, pb_ref, g_ref, da_ref, db_ref, b_ref, w_ref, o_ref):
    p = lax.rsqrt(da_ref[:, 0:1] + db_ref[:, 0:1] + 1.0)
    t = p * (pa_ref[...] + pb_ref[...] + g_ref[...]) + b_ref[...]
    h = jnp.maximum(t, 0.0)
    o_ref[...] = p * jnp.dot(h, w_ref[...], preferred_element_type=jnp.float32)


def _tc3_body(pa_ref, pb_ref, g_ref, da_ref, db_ref, b_ref, wa_ref, ba_ref, o_ref):
    p = lax.rsqrt(da_ref[:, 0:1] + db_ref[:, 0:1] + 1.0)
    h = p * (pa_ref[...] + pb_ref[...] + g_ref[...]) + b_ref[...]
    logits = jnp.dot(h, wa_ref[...], preferred_element_type=jnp.float32) + ba_ref[...]
    m = jnp.max(logits, axis=1, keepdims=True)
    e = jnp.exp(logits - m)
    o_ref[...] = e / jnp.sum(e, axis=1, keepdims=True)


_BLK = 1000
_GRID = N_NODES // _BLK


def _row_blk(shape_minor):
    nmin = len(shape_minor)
    return pl.BlockSpec((_BLK,) + shape_minor, lambda i: (i,) + (0,) * nmin)


def _full(shape):
    n = len(shape)
    return pl.BlockSpec(shape, lambda i: (0,) * n)


def _tc1(x, W1, da, db):
    return pl.pallas_call(
        _tc1_body,
        grid=(_GRID,),
        in_specs=[
            _row_blk((D_FEAT,)),
            _full((D_FEAT, D_FEAT)),
            _row_blk((DEG_W,)),
            _row_blk((DEG_W,)),
        ],
        out_specs=_row_blk((D_FEAT,)),
        out_shape=jax.ShapeDtypeStruct((N_NODES, D_FEAT), jnp.float32),
    )(x, W1, da, db)


def _tc2(pa, pb, g, da, db, b, W):
    return pl.pallas_call(
        _tc2_body,
        grid=(_GRID,),
        in_specs=[
            _row_blk((D_FEAT,)),
            _row_blk((D_FEAT,)),
            _row_blk((D_FEAT,)),
            _row_blk((DEG_W,)),
            _row_blk((DEG_W,)),
            _full((1, D_FEAT)),
            _full((D_FEAT, D_FEAT)),
        ],
        out_specs=_row_blk((D_FEAT,)),
        out_shape=jax.ShapeDtypeStruct((N_NODES, D_FEAT), jnp.float32),
    )(pa, pb, g, da, db, b, W)


def _tc3(pa, pb, g, da, db, b, Wa, ba):
    return pl.pallas_call(
        _tc3_body,
        grid=(_GRID,),
        in_specs=[
            _row_blk((D_FEAT,)),
            _row_blk((D_FEAT,)),
            _row_blk((D_FEAT,)),
            _row_blk((DEG_W,)),
            _row_blk((DEG_W,)),
            _full((1, D_FEAT)),
            _full((D_FEAT, OUT_NODES)),
            _full((1, OUT_NODES)),
        ],
        out_specs=_row_blk((OUT_NODES,)),
        out_shape=jax.ShapeDtypeStruct((N_NODES, OUT_NODES), jnp.float32),
    )(pa, pb, g, da, db, b, Wa, ba)


@jax.jit
def kernel(x, edge_index, W1, b1, W2, b2, Wa, ba):
    ei = edge_index.astype(jnp.int32)
    pad_s = jnp.zeros((E_PAD - N_EDGES,), jnp.int32)
    pad_d = DUMMY + jnp.arange(E_PAD - N_EDGES, dtype=jnp.int32) % (ACC_ROWS - DUMMY)
    src3 = jnp.concatenate([ei[0], pad_s]).reshape(NW, CPW, CHUNK)
    dst3 = jnp.concatenate([ei[1], pad_d]).reshape(NW, CPW, CHUNK)

    deg_parts = _make_deg_kernel()(dst3)
    da = deg_parts[0, :N_NODES, :DEG_W]
    db = deg_parts[1, :N_NODES, :DEG_W]

    agg = _make_agg_kernel()

    g1 = _tc1(x, W1, da, db)
    parts1 = agg(g1, src3, dst3)
    g2 = _tc2(parts1[0, :N_NODES], parts1[1, :N_NODES], g1, da, db,
              b1.reshape(1, D_FEAT), W2)
    parts2 = agg(g2, src3, dst3)
    s = _tc3(parts2[0, :N_NODES], parts2[1, :N_NODES], g2, da, db,
             b2.reshape(1, D_FEAT), Wa, ba.reshape(1, OUT_NODES))
    return s


# speedup vs baseline: 2.4920x; 2.4920x over previous
import jax
import jax.numpy as jnp
from jax import lax
from jax.experimental import pallas as pl
from jax.experimental.pallas import tpu as pltpu
from jax.experimental.pallas import tpu_sc as plsc

N_NODES = 10000
D_FEAT = 128
OUT_NODES = 100
N_EDGES = 320000

NC = 2
NS = 16
NW = NC * NS
CHUNK = 128
EPW = (N_EDGES + NW - 1) // NW
CPW = 80
HALF = CPW // 2
EPW_PAD = CPW * CHUNK
E_PAD = NW * EPW_PAD
ACC_ROWS = NS * 640
DUMMY = N_NODES
RPT = ACC_ROWS // NS
DEG_W = 8


def _sc_mesh():
    return plsc.VectorSubcoreMesh(
        core_axis_name="c", subcore_axis_name="s", num_cores=NC, num_subcores=NS
    )


def _deg_body(dst_hbm, out_hbm, dst_v, ones_v, acc, sem):
    c = lax.axis_index("c")
    s = lax.axis_index("s")
    wid = c * NS + s

    zeros16 = jnp.zeros((16,), jnp.float32)
    ones16 = jnp.ones((16,), jnp.float32)

    @pl.loop(0, CHUNK)
    def _(r):
        for k in range(D_FEAT // 16):
            ones_v[r, pl.ds(k * 16, 16)] = zeros16

    pltpu.sync_copy(dst_hbm.at[wid], dst_v)
    for h in range(RPT // CHUNK):
        pltpu.sync_copy(ones_v, acc.at[pl.ds(s * RPT + h * CHUNK, CHUNK)])

    @pl.loop(0, CHUNK)
    def _(r):
        for k in range(D_FEAT // 16):
            ones_v[r, pl.ds(k * 16, 16)] = ones16

    plsc.subcore_barrier()

    @pl.loop(0, 8)
    def _(j):
        pltpu.async_copy(ones_v, acc.at[dst_v.at[j]], sem, add=True)

    @pl.loop(8, CPW)
    def _(j):
        pltpu.async_copy(ones_v, acc.at[dst_v.at[j]], sem, add=True)
        pltpu.make_async_copy(out_hbm.at[c, pl.ds(0, CHUNK)], ones_v, sem).wait()

    @pl.loop(0, 8)
    def _(j):
        pltpu.make_async_copy(out_hbm.at[c, pl.ds(0, CHUNK)], ones_v, sem).wait()

    plsc.subcore_barrier()
    for h in range(RPT // CHUNK):
        pltpu.sync_copy(acc.at[pl.ds(s * RPT + h * CHUNK, CHUNK)], ones_v)
        pltpu.sync_copy(ones_v, out_hbm.at[c, pl.ds(s * RPT + h * CHUNK, CHUNK)])


def _make_deg_kernel():
    return pl.kernel(
        _deg_body,
        out_type=jax.ShapeDtypeStruct((NC, ACC_ROWS, D_FEAT), jnp.float32),
        mesh=_sc_mesh(),
        scratch_types=[
            pltpu.VMEM((CPW, CHUNK), jnp.int32),
            pltpu.VMEM((CHUNK, D_FEAT), jnp.float32),
            pltpu.VMEM_SHARED((ACC_ROWS, D_FEAT), jnp.float32),
            pltpu.SemaphoreType.DMA,
        ],
    )


def _agg_body(g_hbm, src_hbm, dst_hbm, out_hbm, src_v, dst_v, ra, rb,
              acc, gsa, gsb, ssa, ssb):
    c = lax.axis_index("c")
    s = lax.axis_index("s")
    wid = c * NS + s

    zeros16 = jnp.zeros((16,), jnp.float32)

    @pl.loop(0, CHUNK)
    def _(r):
        for k in range(D_FEAT // 16):
            ra[r, pl.ds(k * 16, 16)] = zeros16

    for h in range(RPT // CHUNK):
        pltpu.sync_copy(ra, acc.at[pl.ds(s * RPT + h * CHUNK, CHUNK)])
    plsc.subcore_barrier()

    def wait_gather(buf, sem):
        pltpu.make_async_copy(g_hbm.at[src_v.at[0]], buf, sem).wait()

    def wait_scatter(buf, sem):
        pltpu.make_async_copy(g_hbm.at[src_v.at[0]], buf, sem).wait()

    for half in range(2):
        pltpu.sync_copy(src_hbm.at[wid, pl.ds(half * HALF, HALF)], src_v)
        pltpu.sync_copy(dst_hbm.at[wid, pl.ds(half * HALF, HALF)], dst_v)

        @pl.loop(0, HALF)
        def _(j):
            pltpu.async_copy(g_hbm.at[src_v.at[j]], ra, gsa).wait()
            pltpu.async_copy(ra, acc.at[dst_v.at[j]], ssa, add=True).wait()

    plsc.subcore_barrier()
    for h in range(RPT // CHUNK):
        pltpu.sync_copy(acc.at[pl.ds(s * RPT + h * CHUNK, CHUNK)], ra)
        pltpu.sync_copy(ra, out_hbm.at[c, pl.ds(s * RPT + h * CHUNK, CHUNK)])


def _make_agg_kernel():
    return pl.kernel(
        _agg_body,
        out_type=jax.ShapeDtypeStruct((NC, ACC_ROWS, D_FEAT), jnp.float32),
        mesh=_sc_mesh(),
        scratch_types=[
            pltpu.VMEM((HALF, CHUNK), jnp.int32),
            pltpu.VMEM((HALF, CHUNK), jnp.int32),
            pltpu.VMEM((CHUNK, D_FEAT), jnp.float32),
            pltpu.VMEM((CHUNK, D_FEAT), jnp.float32),
            pltpu.VMEM_SHARED((ACC_ROWS, D_FEAT), jnp.float32),
            pltpu.SemaphoreType.DMA,
            pltpu.SemaphoreType.DMA,
            pltpu.SemaphoreType.DMA,
            pltpu.SemaphoreType.DMA,
        ],
    )


def _tc1_body(x_ref, w_ref, da_ref, db_ref, g_ref):
    p = lax.rsqrt(da_ref[:, 0:1] + db_ref[:, 0:1] + 1.0)
    h = jnp.dot(x_ref[...], w_ref[...], preferred_element_type=jnp.float32)
    g_ref[...] = p * h


def _tc2_body(pa_ref, pb_ref, g_ref, da_ref, db_ref, b_ref, w_ref, o_ref):
    p = lax.rsqrt(da_ref[:, 0:1] + db_ref[:, 0:1] + 1.0)
    t = p * (pa_ref[...] + pb_ref[...] + g_ref[...]) + b_ref[...]
    h = jnp.maximum(t, 0.0)
    o_ref[...] = p * jnp.dot(h, w_ref[...], preferred_element_type=jnp.float32)


def _tc3_body(pa_ref, pb_ref, g_ref, da_ref, db_ref, b_ref, wa_ref, ba_ref, o_ref):
    p = lax.rsqrt(da_ref[:, 0:1] + db_ref[:, 0:1] + 1.0)
    h = p * (pa_ref[...] + pb_ref[...] + g_ref[...]) + b_ref[...]
    logits = jnp.dot(h, wa_ref[...], preferred_element_type=jnp.float32) + ba_ref[...]
    m = jnp.max(logits, axis=1, keepdims=True)
    e = jnp.exp(logits - m)
    o_ref[...] = e / jnp.sum(e, axis=1, keepdims=True)


_BLK = 1000
_GRID = N_NODES // _BLK


def _row_blk(shape_minor):
    nmin = len(shape_minor)
    return pl.BlockSpec((_BLK,) + shape_minor, lambda i: (i,) + (0,) * nmin)


def _full(shape):
    n = len(shape)
    return pl.BlockSpec(shape, lambda i: (0,) * n)


def _tc1(x, W1, da, db):
    return pl.pallas_call(
        _tc1_body,
        grid=(_GRID,),
        in_specs=[
            _row_blk((D_FEAT,)),
            _full((D_FEAT, D_FEAT)),
            _row_blk((DEG_W,)),
            _row_blk((DEG_W,)),
        ],
        out_specs=_row_blk((D_FEAT,)),
        out_shape=jax.ShapeDtypeStruct((N_NODES, D_FEAT), jnp.float32),
    )(x, W1, da, db)


def _tc2(pa, pb, g, da, db, b, W):
    return pl.pallas_call(
        _tc2_body,
        grid=(_GRID,),
        in_specs=[
            _row_blk((D_FEAT,)),
            _row_blk((D_FEAT,)),
            _row_blk((D_FEAT,)),
            _row_blk((DEG_W,)),
            _row_blk((DEG_W,)),
            _full((1, D_FEAT)),
            _full((D_FEAT, D_FEAT)),
        ],
        out_specs=_row_blk((D_FEAT,)),
        out_shape=jax.ShapeDtypeStruct((N_NODES, D_FEAT), jnp.float32),
    )(pa, pb, g, da, db, b, W)


def _tc3(pa, pb, g, da, db, b, Wa, ba):
    return pl.pallas_call(
        _tc3_body,
        grid=(_GRID,),
        in_specs=[
            _row_blk((D_FEAT,)),
            _row_blk((D_FEAT,)),
            _row_blk((D_FEAT,)),
            _row_blk((DEG_W,)),
            _row_blk((DEG_W,)),
            _full((1, D_FEAT)),
            _full((D_FEAT, OUT_NODES)),
            _full((1, OUT_NODES)),
        ],
        out_specs=_row_blk((OUT_NODES,)),
        out_shape=jax.ShapeDtypeStruct((N_NODES, OUT_NODES), jnp.float32),
    )(pa, pb, g, da, db, b, Wa, ba)


@jax.jit
def kernel(x, edge_index, W1, b1, W2, b2, Wa, ba):
    ei = edge_index.astype(jnp.int32)
    pad_s = jnp.arange(E_PAD - N_EDGES, dtype=jnp.int32) % N_NODES
    pad_d = DUMMY + jnp.arange(E_PAD - N_EDGES, dtype=jnp.int32) % (ACC_ROWS - DUMMY)
    src3 = jnp.concatenate([ei[0], pad_s]).reshape(NW, CPW, CHUNK)
    dst3 = jnp.concatenate([ei[1], pad_d]).reshape(NW, CPW, CHUNK)

    deg_parts = _make_deg_kernel()(dst3)
    da = deg_parts[0, :N_NODES, :DEG_W]
    db = deg_parts[1, :N_NODES, :DEG_W]

    agg = _make_agg_kernel()

    g1 = _tc1(x, W1, da, db)
    parts1 = agg(g1, src3, dst3)
    g2 = _tc2(parts1[0, :N_NODES], parts1[1, :N_NODES], g1, da, db,
              b1.reshape(1, D_FEAT), W2)
    parts2 = agg(g2, src3, dst3)
    s = _tc3(parts2[0, :N_NODES], parts2[1, :N_NODES], g2, da, db,
             b2.reshape(1, D_FEAT), Wa, ba.reshape(1, OUT_NODES))
    return s


# speedup vs baseline: 2.8274x; 1.1346x over previous
import jax
import jax.numpy as jnp
from jax import lax
from jax.experimental import pallas as pl
from jax.experimental.pallas import tpu as pltpu
from jax.experimental.pallas import tpu_sc as plsc

N_NODES = 10000
D_FEAT = 128
OUT_NODES = 100
N_EDGES = 320000

NC = 2
NS = 16
NW = NC * NS
CHUNK = 128
EPW = (N_EDGES + NW - 1) // NW
CPW = 80
HALF = CPW // 2
EPW_PAD = CPW * CHUNK
E_PAD = NW * EPW_PAD
ACC_ROWS = NS * 640
DUMMY = N_NODES
RPT = ACC_ROWS // NS
DEG_W = 8


def _sc_mesh():
    return plsc.VectorSubcoreMesh(
        core_axis_name="c", subcore_axis_name="s", num_cores=NC, num_subcores=NS
    )


def _deg_body(dst_hbm, out_hbm, dst_v, ones_v, acc, sem):
    c = lax.axis_index("c")
    s = lax.axis_index("s")
    wid = c * NS + s

    zeros16 = jnp.zeros((16,), jnp.float32)
    ones16 = jnp.ones((16,), jnp.float32)

    @pl.loop(0, CHUNK)
    def _(r):
        for k in range(D_FEAT // 16):
            ones_v[r, pl.ds(k * 16, 16)] = zeros16

    pltpu.sync_copy(dst_hbm.at[wid], dst_v)
    for h in range(RPT // CHUNK):
        pltpu.sync_copy(ones_v, acc.at[pl.ds(s * RPT + h * CHUNK, CHUNK)])

    @pl.loop(0, CHUNK)
    def _(r):
        for k in range(D_FEAT // 16):
            ones_v[r, pl.ds(k * 16, 16)] = ones16

    plsc.subcore_barrier()

    @pl.loop(0, 8)
    def _(j):
        pltpu.async_copy(ones_v, acc.at[dst_v.at[j]], sem, add=True)

    @pl.loop(8, CPW)
    def _(j):
        pltpu.async_copy(ones_v, acc.at[dst_v.at[j]], sem, add=True)
        pltpu.make_async_copy(out_hbm.at[c, pl.ds(0, CHUNK)], ones_v, sem).wait()

    @pl.loop(0, 8)
    def _(j):
        pltpu.make_async_copy(out_hbm.at[c, pl.ds(0, CHUNK)], ones_v, sem).wait()

    plsc.subcore_barrier()
    for h in range(RPT // CHUNK):
        pltpu.sync_copy(acc.at[pl.ds(s * RPT + h * CHUNK, CHUNK)], ones_v)
        pltpu.sync_copy(ones_v, out_hbm.at[c, pl.ds(s * RPT + h * CHUNK, CHUNK)])


def _make_deg_kernel():
    return pl.kernel(
        _deg_body,
        out_type=jax.ShapeDtypeStruct((NC, ACC_ROWS, D_FEAT), jnp.float32),
        mesh=_sc_mesh(),
        scratch_types=[
            pltpu.VMEM((CPW, CHUNK), jnp.int32),
            pltpu.VMEM((CHUNK, D_FEAT), jnp.float32),
            pltpu.VMEM_SHARED((ACC_ROWS, D_FEAT), jnp.float32),
            pltpu.SemaphoreType.DMA,
        ],
    )


def _agg_body(g_hbm, src_hbm, dst_hbm, out_hbm, src_v, dst_v, ra, rb,
              acc, gsa, gsb, ssa, ssb):
    c = lax.axis_index("c")
    s = lax.axis_index("s")
    wid = c * NS + s

    zeros16 = jnp.zeros((16,), jnp.float32)

    @pl.loop(0, CHUNK)
    def _(r):
        for k in range(D_FEAT // 16):
            ra[r, pl.ds(k * 16, 16)] = zeros16

    for h in range(RPT // CHUNK):
        pltpu.sync_copy(ra, acc.at[pl.ds(s * RPT + h * CHUNK, CHUNK)])
    plsc.subcore_barrier()

    def wait_gather(buf, sem):
        pltpu.make_async_copy(g_hbm.at[src_v.at[0]], buf, sem).wait()

    def wait_scatter(buf, sem):
        pltpu.make_async_copy(g_hbm.at[src_v.at[0]], buf, sem).wait()

    for half in range(2):
        pltpu.sync_copy(src_hbm.at[wid, pl.ds(half * HALF, HALF)], src_v)
        pltpu.sync_copy(dst_hbm.at[wid, pl.ds(half * HALF, HALF)], dst_v)
        pltpu.async_copy(g_hbm.at[src_v.at[0]], ra, gsa)
        pltpu.async_copy(g_hbm.at[src_v.at[1]], rb, gsb)

        @pl.loop(0, HALF // 2 - 1)
        def _(t):
            wait_gather(ra, gsa)
            pltpu.async_copy(ra, acc.at[dst_v.at[2 * t]], ssa, add=True)
            wait_gather(rb, gsb)
            pltpu.async_copy(rb, acc.at[dst_v.at[2 * t + 1]], ssb, add=True)
            wait_scatter(ra, ssa)
            pltpu.async_copy(g_hbm.at[src_v.at[2 * t + 2]], ra, gsa)
            wait_scatter(rb, ssb)
            pltpu.async_copy(g_hbm.at[src_v.at[2 * t + 3]], rb, gsb)

        wait_gather(ra, gsa)
        pltpu.async_copy(ra, acc.at[dst_v.at[HALF - 2]], ssa, add=True)
        wait_gather(rb, gsb)
        pltpu.async_copy(rb, acc.at[dst_v.at[HALF - 1]], ssb, add=True)
        wait_scatter(ra, ssa)
        wait_scatter(rb, ssb)

    plsc.subcore_barrier()
    for h in range(RPT // CHUNK):
        pltpu.sync_copy(acc.at[pl.ds(s * RPT + h * CHUNK, CHUNK)], ra)
        pltpu.sync_copy(ra, out_hbm.at[c, pl.ds(s * RPT + h * CHUNK, CHUNK)])


def _make_agg_kernel():
    return pl.kernel(
        _agg_body,
        out_type=jax.ShapeDtypeStruct((NC, ACC_ROWS, D_FEAT), jnp.float32),
        mesh=_sc_mesh(),
        scratch_types=[
            pltpu.VMEM((HALF, CHUNK), jnp.int32),
            pltpu.VMEM((HALF, CHUNK), jnp.int32),
            pltpu.VMEM((CHUNK, D_FEAT), jnp.float32),
            pltpu.VMEM((CHUNK, D_FEAT), jnp.float32),
            pltpu.VMEM_SHARED((ACC_ROWS, D_FEAT), jnp.float32),
            pltpu.SemaphoreType.DMA,
            pltpu.SemaphoreType.DMA,
            pltpu.SemaphoreType.DMA,
            pltpu.SemaphoreType.DMA,
        ],
    )


def _tc1_body(x_ref, w_ref, da_ref, db_ref, g_ref):
    p = lax.rsqrt(da_ref[:, 0:1] + db_ref[:, 0:1] + 1.0)
    h = jnp.dot(x_ref[...], w_ref[...], preferred_element_type=jnp.float32)
    g_ref[...] = p * h


def _tc2_body(pa_ref, pb_ref, g_ref, da_ref, db_ref, b_ref, w_ref, o_ref):
    p = lax.rsqrt(da_ref[:, 0:1] + db_ref[:, 0:1] + 1.0)
    t = p * (pa_ref[...] + pb_ref[...] + g_ref[...]) + b_ref[...]
    h = jnp.maximum(t, 0.0)
    o_ref[...] = p * jnp.dot(h, w_ref[...], preferred_element_type=jnp.float32)


def _tc3_body(pa_ref, pb_ref, g_ref, da_ref, db_ref, b_ref, wa_ref, ba_ref, o_ref):
    p = lax.rsqrt(da_ref[:, 0:1] + db_ref[:, 0:1] + 1.0)
    h = p * (pa_ref[...] + pb_ref[...] + g_ref[...]) + b_ref[...]
    logits = jnp.dot(h, wa_ref[...], preferred_element_type=jnp.float32) + ba_ref[...]
    m = jnp.max(logits, axis=1, keepdims=True)
    e = jnp.exp(logits - m)
    o_ref[...] = e / jnp.sum(e, axis=1, keepdims=True)


_BLK = 1000
_GRID = N_NODES // _BLK


def _row_blk(shape_minor):
    nmin = len(shape_minor)
    return pl.BlockSpec((_BLK,) + shape_minor, lambda i: (i,) + (0,) * nmin)


def _full(shape):
    n = len(shape)
    return pl.BlockSpec(shape, lambda i: (0,) * n)


def _tc1(x, W1, da, db):
    return pl.pallas_call(
        _tc1_body,
        grid=(_GRID,),
        in_specs=[
            _row_blk((D_FEAT,)),
            _full((D_FEAT, D_FEAT)),
            _row_blk((DEG_W,)),
            _row_blk((DEG_W,)),
        ],
        out_specs=_row_blk((D_FEAT,)),
        out_shape=jax.ShapeDtypeStruct((N_NODES, D_FEAT), jnp.float32),
    )(x, W1, da, db)


def _tc2(pa, pb, g, da, db, b, W):
    return pl.pallas_call(
        _tc2_body,
        grid=(_GRID,),
        in_specs=[
            _row_blk((D_FEAT,)),
            _row_blk((D_FEAT,)),
            _row_blk((D_FEAT,)),
            _row_blk((DEG_W,)),
            _row_blk((DEG_W,)),
            _full((1, D_FEAT)),
            _full((D_FEAT, D_FEAT)),
        ],
        out_specs=_row_blk((D_FEAT,)),
        out_shape=jax.ShapeDtypeStruct((N_NODES, D_FEAT), jnp.float32),
    )(pa, pb, g, da, db, b, W)


def _tc3(pa, pb, g, da, db, b, Wa, ba):
    return pl.pallas_call(
        _tc3_body,
        grid=(_GRID,),
        in_specs=[
            _row_blk((D_FEAT,)),
            _row_blk((D_FEAT,)),
            _row_blk((D_FEAT,)),
            _row_blk((DEG_W,)),
            _row_blk((DEG_W,)),
            _full((1, D_FEAT)),
            _full((D_FEAT, OUT_NODES)),
            _full((1, OUT_NODES)),
        ],
        out_specs=_row_blk((OUT_NODES,)),
        out_shape=jax.ShapeDtypeStruct((N_NODES, OUT_NODES), jnp.float32),
    )(pa, pb, g, da, db, b, Wa, ba)


@jax.jit
def kernel(x, edge_index, W1, b1, W2, b2, Wa, ba):
    ei = edge_index.astype(jnp.int32)
    pad_s = jnp.arange(E_PAD - N_EDGES, dtype=jnp.int32) % N_NODES
    pad_d = DUMMY + jnp.arange(E_PAD - N_EDGES, dtype=jnp.int32) % (ACC_ROWS - DUMMY)
    src3 = jnp.concatenate([ei[0], pad_s]).reshape(NW, CPW, CHUNK)
    dst3 = jnp.concatenate([ei[1], pad_d]).reshape(NW, CPW, CHUNK)

    deg_parts = _make_deg_kernel()(dst3)
    da = deg_parts[0, :N_NODES, :DEG_W]
    db = deg_parts[1, :N_NODES, :DEG_W]

    agg = _make_agg_kernel()

    g1 = _tc1(x, W1, da, db)
    parts1 = agg(g1, src3, dst3)
    g2 = _tc2(parts1[0, :N_NODES], parts1[1, :N_NODES], g1, da, db,
              b1.reshape(1, D_FEAT), W2)
    parts2 = agg(g2, src3, dst3)
    s = _tc3(parts2[0, :N_NODES], parts2[1, :N_NODES], g2, da, db,
             b2.reshape(1, D_FEAT), Wa, ba.reshape(1, OUT_NODES))
    return s
